# Initial kernel scaffold; baseline (speedup 1.0000x reference)
import jax
import jax.numpy as jnp
from jax.experimental import pallas as pl


def kernel(x, edge_index, W0, b0, W1, b1, Wf, bf):
    raise NotImplementedError("write your pallas kernel here")


# speedup vs baseline: 5.1915x; 5.1915x over previous
import functools

import jax
import jax.numpy as jnp
from jax import lax
from jax.experimental import pallas as pl
from jax.experimental.pallas import tpu as pltpu
from jax.experimental.pallas import tpu_sc as plsc

_NC = 2
_NS = 16
_NW = _NC * _NS
_CH = 128
_BLK = 1000


def _seg_sum_call(p, srcp, dstp, npad):
    _, w = p.shape
    epad = srcp.shape[0]
    ept = epad // _NW
    nch = ept // _CH
    rpt = npad // _NS
    mesh = plsc.VectorSubcoreMesh(core_axis_name="c", subcore_axis_name="s")

    @functools.partial(
        pl.kernel,
        mesh=mesh,
        out_type=jax.ShapeDtypeStruct((_NC, npad, w), jnp.float32),
        scratch_types=[
            pltpu.VMEM((_CH,), jnp.int32),
            pltpu.VMEM((_CH,), jnp.int32),
            pltpu.VMEM((_CH, w), jnp.float32),
            pltpu.VMEM_SHARED((npad, w), jnp.float32),
            pltpu.SemaphoreType.DMA,
        ],
        compiler_params=pltpu.CompilerParams(
            use_tc_tiling_on_sc=(w % 128 == 0)
        ),
    )
    def k(p_hbm, src_hbm, dst_hbm, out_hbm, sidx, didx, rows, acc, sem):
        cid = lax.axis_index("c")
        sid = lax.axis_index("s")
        wid = cid * _NS + sid
        zero = jnp.zeros((16,), jnp.float32)

        def zrow(r, carry):
            for c in range(w // 16):
                rows[r, pl.ds(c * 16, 16)] = zero
            return carry

        lax.fori_loop(0, _CH, zrow, 0)
        for z in range(rpt // _CH):
            pltpu.sync_copy(rows, acc.at[pl.ds(sid * rpt + z * _CH, _CH)])
        plsc.subcore_barrier()

        def step(i, carry):
            base = pl.multiple_of(wid * ept + i * _CH, _CH)
            pltpu.sync_copy(src_hbm.at[pl.ds(base, _CH)], sidx)
            pltpu.sync_copy(dst_hbm.at[pl.ds(base, _CH)], didx)
            pltpu.async_copy(p_hbm.at[sidx], rows, sem).wait()
            pltpu.sync_copy(rows, acc.at[didx], add=True)
            return carry

        lax.fori_loop(0, nch, step, 0)
        plsc.subcore_barrier()
        pltpu.sync_copy(
            acc.at[pl.ds(sid * rpt, rpt)], out_hbm.at[cid, pl.ds(sid * rpt, rpt)]
        )

    return k(p, srcp, dstp)


def _deg_call(dstp, npad):
    w = 16
    epad = dstp.shape[0]
    ept = epad // _NW
    nch = ept // _CH
    rpt = npad // _NS
    mesh = plsc.VectorSubcoreMesh(core_axis_name="c", subcore_axis_name="s")

    @functools.partial(
        pl.kernel,
        mesh=mesh,
        out_type=jax.ShapeDtypeStruct((_NC, npad, w), jnp.float32),
        scratch_types=[
            pltpu.VMEM((_CH,), jnp.int32),
            pltpu.VMEM((_CH, w), jnp.float32),
            pltpu.VMEM((_CH, w), jnp.float32),
            pltpu.VMEM_SHARED((npad, w), jnp.float32),
        ],
        compiler_params=pltpu.CompilerParams(use_tc_tiling_on_sc=False),
    )
    def k(dst_hbm, out_hbm, didx, ones_v, zrows, acc):
        cid = lax.axis_index("c")
        sid = lax.axis_index("s")
        wid = cid * _NS + sid
        one = jnp.ones((16,), jnp.float32)
        zero = jnp.zeros((16,), jnp.float32)

        def fill(r, carry):
            ones_v[r, pl.ds(0, 16)] = one
            zrows[r, pl.ds(0, 16)] = zero
            return carry

        lax.fori_loop(0, _CH, fill, 0)
        for z in range(rpt // _CH):
            pltpu.sync_copy(zrows, acc.at[pl.ds(sid * rpt + z * _CH, _CH)])
        plsc.subcore_barrier()

        def step(i, carry):
            base = pl.multiple_of(wid * ept + i * _CH, _CH)
            pltpu.sync_copy(dst_hbm.at[pl.ds(base, _CH)], didx)
            pltpu.sync_copy(ones_v, acc.at[didx], add=True)
            return carry

        lax.fori_loop(0, nch, step, 0)
        plsc.subcore_barrier()
        pltpu.sync_copy(
            acc.at[pl.ds(sid * rpt, rpt)], out_hbm.at[cid, pl.ds(sid * rpt, rpt)]
        )

    return k(dstp)


def _tc1_call(x, degp, w0r, w00, wfa, wf0, bfr):
    n, d = x.shape
    hid = w0r.shape[1]
    emb = wfa.shape[1]
    g = n // _BLK

    def body(x_ref, dp_ref, w0r_ref, w00_ref, wfa_ref, wf0_ref, bf_ref, p1_ref, oa_ref):
        deg = dp_ref[0, :, 0:1] + dp_ref[1, :, 0:1]
        xb = x_ref[...]
        p1_ref[...] = jnp.dot(xb, w0r_ref[...]) + deg * w00_ref[...]
        oa_ref[...] = jnp.dot(xb, wfa_ref[...]) + deg * wf0_ref[...] + bf_ref[...]

    return pl.pallas_call(
        body,
        grid=(g,),
        in_specs=[
            pl.BlockSpec((_BLK, d), lambda i: (i, 0)),
            pl.BlockSpec((_NC, _BLK, 16), lambda i: (0, i, 0)),
            pl.BlockSpec((d, hid), lambda i: (0, 0)),
            pl.BlockSpec((1, hid), lambda i: (0, 0)),
            pl.BlockSpec((d, emb), lambda i: (0, 0)),
            pl.BlockSpec((1, emb), lambda i: (0, 0)),
            pl.BlockSpec((1, emb), lambda i: (0, 0)),
        ],
        out_specs=[
            pl.BlockSpec((_BLK, hid), lambda i: (i, 0)),
            pl.BlockSpec((_BLK, emb), lambda i: (i, 0)),
        ],
        out_shape=[
            jax.ShapeDtypeStruct((n, hid), jnp.float32),
            jax.ShapeDtypeStruct((n, emb), jnp.float32),
        ],
    )(x, degp, w0r, w00, wfa, wf0, bfr)


def _tc2_call(s1p, degp, p1, b0r, w1, wfb, oa):
    n, hid = p1.shape
    emb = w1.shape[1]
    g = n // _BLK

    def body(s_ref, dp_ref, p1_ref, b0_ref, w1_ref, wfb_ref, oa_ref, p2_ref, o2_ref):
        deg = dp_ref[0, :, 0:1] + dp_ref[1, :, 0:1]
        s = s_ref[0] + s_ref[1]
        mean = s / jnp.maximum(deg, 1.0)
        agg = jnp.where(deg > 0.0, mean, p1_ref[...])
        h2 = jnp.maximum(agg + b0_ref[...], 0.0)
        p2_ref[...] = jnp.dot(h2, w1_ref[...])
        o2_ref[...] = oa_ref[...] + jnp.dot(h2, wfb_ref[...])

    return pl.pallas_call(
        body,
        grid=(g,),
        in_specs=[
            pl.BlockSpec((_NC, _BLK, hid), lambda i: (0, i, 0)),
            pl.BlockSpec((_NC, _BLK, 16), lambda i: (0, i, 0)),
            pl.BlockSpec((_BLK, hid), lambda i: (i, 0)),
            pl.BlockSpec((1, hid), lambda i: (0, 0)),
            pl.BlockSpec((hid, emb), lambda i: (0, 0)),
            pl.BlockSpec((hid, emb), lambda i: (0, 0)),
            pl.BlockSpec((_BLK, emb), lambda i: (i, 0)),
        ],
        out_specs=[
            pl.BlockSpec((_BLK, emb), lambda i: (i, 0)),
            pl.BlockSpec((_BLK, emb), lambda i: (i, 0)),
        ],
        out_shape=[
            jax.ShapeDtypeStruct((n, emb), jnp.float32),
            jax.ShapeDtypeStruct((n, emb), jnp.float32),
        ],
    )(s1p, degp, p1, b0r, w1, wfb, oa)


def _tc3_call(s2p, degp, p2, b1r, wfc, o2):
    n, emb = p2.shape
    g = n // _BLK

    def body(s_ref, dp_ref, p2_ref, b1_ref, wfc_ref, o2_ref, out_ref):
        deg = dp_ref[0, :, 0:1] + dp_ref[1, :, 0:1]
        s = s_ref[0] + s_ref[1]
        mean = s / jnp.maximum(deg, 1.0)
        agg = jnp.where(deg > 0.0, mean, p2_ref[...])
        h3 = jnp.maximum(agg + b1_ref[...], 0.0)
        out_ref[...] = o2_ref[...] + jnp.dot(h3, wfc_ref[...])

    return pl.pallas_call(
        body,
        grid=(g,),
        in_specs=[
            pl.BlockSpec((_NC, _BLK, emb), lambda i: (0, i, 0)),
            pl.BlockSpec((_NC, _BLK, 16), lambda i: (0, i, 0)),
            pl.BlockSpec((_BLK, emb), lambda i: (i, 0)),
            pl.BlockSpec((1, emb), lambda i: (0, 0)),
            pl.BlockSpec((emb, emb), lambda i: (0, 0)),
            pl.BlockSpec((_BLK, emb), lambda i: (i, 0)),
        ],
        out_specs=pl.BlockSpec((_BLK, emb), lambda i: (i, 0)),
        out_shape=jax.ShapeDtypeStruct((n, emb), jnp.float32),
    )(s2p, degp, p2, b1r, wfc, o2)


def kernel(x, edge_index, W0, b0, W1, b1, Wf, bf):
    n, d = x.shape
    e = edge_index.shape[1]
    hid = W0.shape[1]
    emb = W1.shape[1]

    grain = _NS * _CH
    npad = -(-(n + 1) // grain) * grain
    egrain = _NW * _CH
    epad = -(-e // egrain) * egrain

    src = edge_index[0]
    dst = edge_index[1]
    srcp = jnp.concatenate([src, jnp.zeros((epad - e,), jnp.int32)])
    dstp = jnp.concatenate([dst, jnp.full((epad - e,), n, jnp.int32)])

    degp = _deg_call(dstp, npad)

    p1, oa = _tc1_call(
        x,
        degp,
        W0[1:],
        W0[0:1],
        Wf[1 : d + 1],
        Wf[0:1],
        bf.reshape(1, emb),
    )

    s1p = _seg_sum_call(p1, srcp, dstp, npad)
    p2, o2 = _tc2_call(
        s1p, degp, p1, b0.reshape(1, hid), W1, Wf[d + 1 : d + 1 + hid], oa
    )

    s2p = _seg_sum_call(p2, srcp, dstp, npad)
    out = _tc3_call(s2p, degp, p2, b1.reshape(1, emb), Wf[d + 1 + hid :], o2)
    return out


# speedup vs baseline: 5.6707x; 1.0923x over previous
import functools

import jax
import jax.numpy as jnp
from jax import lax
from jax.experimental import pallas as pl
from jax.experimental.pallas import tpu as pltpu
from jax.experimental.pallas import tpu_sc as plsc

_NC = 2
_NS = 16
_NW = _NC * _NS
_CH = 128
_BLK = 1000


def _seg_sum_call(p, src3, dst3, npad):
    _, w = p.shape
    _, nch, _ = src3.shape
    rpt = npad // _NS
    mesh = plsc.VectorSubcoreMesh(core_axis_name="c", subcore_axis_name="s")

    @functools.partial(
        pl.kernel,
        mesh=mesh,
        out_type=jax.ShapeDtypeStruct((_NC, npad, w), jnp.float32),
        scratch_types=[
            pltpu.VMEM((_CH,), jnp.int32),
            pltpu.VMEM((_CH,), jnp.int32),
            pltpu.VMEM((_CH,), jnp.int32),
            pltpu.VMEM((_CH,), jnp.int32),
            pltpu.VMEM((_CH, w), jnp.float32),
            pltpu.VMEM((_CH, w), jnp.float32),
            pltpu.VMEM_SHARED((npad, w), jnp.float32),
            pltpu.SemaphoreType.DMA,
            pltpu.SemaphoreType.DMA,
            pltpu.SemaphoreType.DMA,
            pltpu.SemaphoreType.DMA,
        ],
        compiler_params=pltpu.CompilerParams(
            use_tc_tiling_on_sc=(w % 128 == 0)
        ),
    )
    def k(p_hbm, src_hbm, dst_hbm, out_hbm, sidx0, sidx1, didx0, didx1,
          rows0, rows1, acc, gs0, gs1, fs0, fs1):
        cid = lax.axis_index("c")
        sid = lax.axis_index("s")
        wid = cid * _NS + sid
        sidx = (sidx0, sidx1)
        didx = (didx0, didx1)
        rows = (rows0, rows1)
        gs = (gs0, gs1)
        fs = (fs0, fs1)
        zero = jnp.zeros((16,), jnp.float32)

        pltpu.async_copy(src_hbm.at[wid, 0], sidx0, fs0)
        pltpu.async_copy(dst_hbm.at[wid, 0], didx0, fs0)
        pltpu.async_copy(src_hbm.at[wid, 1], sidx1, fs1)
        pltpu.async_copy(dst_hbm.at[wid, 1], didx1, fs1)

        def zrow(r, carry):
            for c in range(w // 16):
                rows0[r, pl.ds(c * 16, 16)] = zero
            return carry

        lax.fori_loop(0, _CH, zrow, 0)
        for z in range(rpt // _CH):
            pltpu.sync_copy(rows0, acc.at[pl.ds(sid * rpt + z * _CH, _CH)])
        pltpu.make_async_copy(src_hbm.at[wid, 0], sidx0, fs0).wait()
        pltpu.make_async_copy(dst_hbm.at[wid, 0], didx0, fs0).wait()
        pltpu.async_copy(p_hbm.at[sidx0], rows0, gs0)
        plsc.subcore_barrier()

        def pair(i, carry):
            for b in range(2):
                j = 2 * i + b
                o = 1 - b
                pltpu.make_async_copy(p_hbm.at[sidx[b]], rows[b], gs[b]).wait()

                @pl.when(j + 1 < nch)
                def _():
                    pltpu.make_async_copy(src_hbm.at[wid, 0], sidx[o], fs[o]).wait()
                    pltpu.make_async_copy(dst_hbm.at[wid, 0], didx[o], fs[o]).wait()
                    pltpu.async_copy(p_hbm.at[sidx[o]], rows[o], gs[o])

                pltpu.sync_copy(rows[b], acc.at[didx[b]], add=True)

                @pl.when(j + 2 < nch)
                def _():
                    pltpu.async_copy(src_hbm.at[wid, j + 2], sidx[b], fs[b])
                    pltpu.async_copy(dst_hbm.at[wid, j + 2], didx[b], fs[b])
            return carry

        lax.fori_loop(0, nch // 2, pair, 0)
        plsc.subcore_barrier()
        pltpu.sync_copy(
            acc.at[pl.ds(sid * rpt, rpt)], out_hbm.at[cid, pl.ds(sid * rpt, rpt)]
        )

    return k(p, src3, dst3)


def _deg_call(dst3, npad):
    w = 16
    _, nch, _ = dst3.shape
    rpt = npad // _NS
    mesh = plsc.VectorSubcoreMesh(core_axis_name="c", subcore_axis_name="s")

    @functools.partial(
        pl.kernel,
        mesh=mesh,
        out_type=jax.ShapeDtypeStruct((_NC, npad, w), jnp.float32),
        scratch_types=[
            pltpu.VMEM((nch, _CH), jnp.int32),
            pltpu.VMEM((_CH, w), jnp.float32),
            pltpu.VMEM((_CH, w), jnp.float32),
            pltpu.VMEM_SHARED((npad, w), jnp.float32),
            pltpu.SemaphoreType.DMA,
        ],
        compiler_params=pltpu.CompilerParams(use_tc_tiling_on_sc=False),
    )
    def k(dst_hbm, out_hbm, didx, ones_v, zrows, acc, sem):
        cid = lax.axis_index("c")
        sid = lax.axis_index("s")
        wid = cid * _NS + sid
        one = jnp.ones((16,), jnp.float32)
        zero = jnp.zeros((16,), jnp.float32)

        pltpu.sync_copy(dst_hbm.at[wid], didx)

        def fill(r, carry):
            ones_v[r, pl.ds(0, 16)] = one
            zrows[r, pl.ds(0, 16)] = zero
            return carry

        lax.fori_loop(0, _CH, fill, 0)
        for z in range(rpt // _CH):
            pltpu.sync_copy(zrows, acc.at[pl.ds(sid * rpt + z * _CH, _CH)])
        plsc.subcore_barrier()

        pltpu.async_copy(ones_v, acc.at[didx.at[0]], sem, add=True)

        def step(i, carry):
            pltpu.async_copy(ones_v, acc.at[didx.at[i + 1]], sem, add=True)
            pltpu.make_async_copy(ones_v, acc.at[didx.at[i]], sem).wait()
            return carry

        lax.fori_loop(0, nch - 1, step, 0)
        pltpu.make_async_copy(ones_v, acc.at[didx.at[nch - 1]], sem).wait()
        plsc.subcore_barrier()
        pltpu.sync_copy(
            acc.at[pl.ds(sid * rpt, rpt)], out_hbm.at[cid, pl.ds(sid * rpt, rpt)]
        )

    return k(dst3)


def _tc1_call(x, degp, w0r, w00, wfa, wf0, bfr):
    n, d = x.shape
    hid = w0r.shape[1]
    emb = wfa.shape[1]
    g = n // _BLK

    def body(x_ref, dp_ref, w0r_ref, w00_ref, wfa_ref, wf0_ref, bf_ref, p1_ref, oa_ref):
        deg = dp_ref[0, :, 0:1] + dp_ref[1, :, 0:1]
        xb = x_ref[...]
        p1_ref[...] = jnp.dot(xb, w0r_ref[...]) + deg * w00_ref[...]
        oa_ref[...] = jnp.dot(xb, wfa_ref[...]) + deg * wf0_ref[...] + bf_ref[...]

    return pl.pallas_call(
        body,
        grid=(g,),
        in_specs=[
            pl.BlockSpec((_BLK, d), lambda i: (i, 0)),
            pl.BlockSpec((_NC, _BLK, 16), lambda i: (0, i, 0)),
            pl.BlockSpec((d, hid), lambda i: (0, 0)),
            pl.BlockSpec((1, hid), lambda i: (0, 0)),
            pl.BlockSpec((d, emb), lambda i: (0, 0)),
            pl.BlockSpec((1, emb), lambda i: (0, 0)),
            pl.BlockSpec((1, emb), lambda i: (0, 0)),
        ],
        out_specs=[
            pl.BlockSpec((_BLK, hid), lambda i: (i, 0)),
            pl.BlockSpec((_BLK, emb), lambda i: (i, 0)),
        ],
        out_shape=[
            jax.ShapeDtypeStruct((n, hid), jnp.float32),
            jax.ShapeDtypeStruct((n, emb), jnp.float32),
        ],
    )(x, degp, w0r, w00, wfa, wf0, bfr)


def _tc2_call(s1p, degp, p1, b0r, w1, wfb, oa):
    n, hid = p1.shape
    emb = w1.shape[1]
    g = n // _BLK

    def body(s_ref, dp_ref, p1_ref, b0_ref, w1_ref, wfb_ref, oa_ref, p2_ref, o2_ref):
        deg = dp_ref[0, :, 0:1] + dp_ref[1, :, 0:1]
        s = s_ref[0] + s_ref[1]
        mean = s / jnp.maximum(deg, 1.0)
        agg = jnp.where(deg > 0.0, mean, p1_ref[...])
        h2 = jnp.maximum(agg + b0_ref[...], 0.0)
        p2_ref[...] = jnp.dot(h2, w1_ref[...])
        o2_ref[...] = oa_ref[...] + jnp.dot(h2, wfb_ref[...])

    return pl.pallas_call(
        body,
        grid=(g,),
        in_specs=[
            pl.BlockSpec((_NC, _BLK, hid), lambda i: (0, i, 0)),
            pl.BlockSpec((_NC, _BLK, 16), lambda i: (0, i, 0)),
            pl.BlockSpec((_BLK, hid), lambda i: (i, 0)),
            pl.BlockSpec((1, hid), lambda i: (0, 0)),
            pl.BlockSpec((hid, emb), lambda i: (0, 0)),
            pl.BlockSpec((hid, emb), lambda i: (0, 0)),
            pl.BlockSpec((_BLK, emb), lambda i: (i, 0)),
        ],
        out_specs=[
            pl.BlockSpec((_BLK, emb), lambda i: (i, 0)),
            pl.BlockSpec((_BLK, emb), lambda i: (i, 0)),
        ],
        out_shape=[
            jax.ShapeDtypeStruct((n, emb), jnp.float32),
            jax.ShapeDtypeStruct((n, emb), jnp.float32),
        ],
    )(s1p, degp, p1, b0r, w1, wfb, oa)


def _tc3_call(s2p, degp, p2, b1r, wfc, o2):
    n, emb = p2.shape
    g = n // _BLK

    def body(s_ref, dp_ref, p2_ref, b1_ref, wfc_ref, o2_ref, out_ref):
        deg = dp_ref[0, :, 0:1] + dp_ref[1, :, 0:1]
        s = s_ref[0] + s_ref[1]
        mean = s / jnp.maximum(deg, 1.0)
        agg = jnp.where(deg > 0.0, mean, p2_ref[...])
        h3 = jnp.maximum(agg + b1_ref[...], 0.0)
        out_ref[...] = o2_ref[...] + jnp.dot(h3, wfc_ref[...])

    return pl.pallas_call(
        body,
        grid=(g,),
        in_specs=[
            pl.BlockSpec((_NC, _BLK, emb), lambda i: (0, i, 0)),
            pl.BlockSpec((_NC, _BLK, 16), lambda i: (0, i, 0)),
            pl.BlockSpec((_BLK, emb), lambda i: (i, 0)),
            pl.BlockSpec((1, emb), lambda i: (0, 0)),
            pl.BlockSpec((emb, emb), lambda i: (0, 0)),
            pl.BlockSpec((_BLK, emb), lambda i: (i, 0)),
        ],
        out_specs=pl.BlockSpec((_BLK, emb), lambda i: (i, 0)),
        out_shape=jax.ShapeDtypeStruct((n, emb), jnp.float32),
    )(s2p, degp, p2, b1r, wfc, o2)


def kernel(x, edge_index, W0, b0, W1, b1, Wf, bf):
    n, d = x.shape
    e = edge_index.shape[1]
    hid = W0.shape[1]
    emb = W1.shape[1]

    grain = _NS * _CH
    npad = -(-(n + 1) // grain) * grain
    egrain = _NW * _CH * 2
    epad = -(-e // egrain) * egrain
    nch = epad // (_NW * _CH)

    src = edge_index[0]
    dst = edge_index[1]
    srcp = jnp.concatenate([src, jnp.zeros((epad - e,), jnp.int32)])
    dstp = jnp.concatenate([dst, jnp.full((epad - e,), n, jnp.int32)])
    src3 = srcp.reshape(_NW, nch, _CH)
    dst3 = dstp.reshape(_NW, nch, _CH)

    degp = _deg_call(dst3, npad)

    p1, oa = _tc1_call(
        x,
        degp,
        W0[1:],
        W0[0:1],
        Wf[1 : d + 1],
        Wf[0:1],
        bf.reshape(1, emb),
    )

    s1p = _seg_sum_call(p1, src3, dst3, npad)
    p2, o2 = _tc2_call(
        s1p, degp, p1, b0.reshape(1, hid), W1, Wf[d + 1 : d + 1 + hid], oa
    )

    s2p = _seg_sum_call(p2, src3, dst3, npad)
    out = _tc3_call(s2p, degp, p2, b1.reshape(1, emb), Wf[d + 1 + hid :], o2)
    return out
